# baseline (speedup 1.0000x reference)
import functools

import jax
import jax.numpy as jnp
from jax import lax
from jax.experimental import pallas as pl
from jax.experimental.pallas import tpu as pltpu
from jax.experimental.pallas import tpu_sc as plsc

EPS = 1e-7
BIG = 3e38


def _nn_body(xyz1t_ref, xyz2_ref, iw_ref):
    b = pl.program_id(0)
    x1t = xyz1t_ref[0]
    sq1 = jnp.sum(x1t * x1t, axis=0, keepdims=True)
    x2 = xyz2_ref[0]
    sq2 = jnp.sum(x2 * x2, axis=1, keepdims=True)
    inner = jnp.dot(x2, x1t, preferred_element_type=jnp.float32)
    t = sq1 - 2.0 * inner

    m1 = jnp.min(t, axis=1, keepdims=True)
    t2 = jnp.where(t == m1, BIG, t)
    m2 = jnp.min(t2, axis=1, keepdims=True)
    m3 = jnp.min(jnp.where(t2 == m2, BIG, t2), axis=1, keepdims=True)

    d = [jnp.maximum(v + sq2, EPS) for v in (m1, m2, m3)]
    w = [1.0 / dk for dk in d]
    norm = w[0] + w[1] + w[2]
    w = [wk / norm for wk in w]

    iotacol = lax.broadcasted_iota(
        jnp.int32, (t.shape[1], 1), 0).astype(jnp.float32)
    zero = jnp.zeros_like(t)
    base = jnp.float32(1024.0) * b.astype(jnp.float32)
    idx = []
    for mk in (m1, m2, m3):
        ek = jnp.where(t == mk, 1.0, zero)
        ik = jnp.dot(ek, iotacol, preferred_element_type=jnp.float32,
                     precision=jax.lax.Precision.HIGHEST)
        idx.append(jnp.minimum(ik, 1023.0) + base)
    iw_ref[0] = jnp.concatenate(idx + w + [jnp.zeros_like(m1)] * 2, axis=1)


@jax.jit
def _three_nn_tc(xyz1t, xyz2, blk=2048):
    B, N2, _ = xyz2.shape
    N1 = xyz1t.shape[2]
    return pl.pallas_call(
        _nn_body,
        grid=(B, N2 // blk),
        in_specs=[
            pl.BlockSpec((1, 3, N1), lambda b, j: (b, 0, 0)),
            pl.BlockSpec((1, blk, 3), lambda b, j: (b, j, 0)),
        ],
        out_specs=pl.BlockSpec((1, blk, 8), lambda b, j: (b, j, 0)),
        out_shape=jax.ShapeDtypeStruct((B, N2, 8), jnp.float32),
        compiler_params=pltpu.CompilerParams(
            dimension_semantics=("arbitrary", "arbitrary"),
        ),
    )(xyz1t, xyz2)


_CH = 8
_PTS_PER_W = 1024


def _make_sc_interp(n_points, D):
    info = plsc.get_sparse_core_info()
    NC, NS = info.num_cores, info.num_subcores
    nw = NC * NS
    ppw = n_points // nw
    nch = ppw // _CH
    mesh = plsc.VectorSubcoreMesh(core_axis_name="c", subcore_axis_name="s")

    @functools.partial(
        pl.kernel, mesh=mesh,
        out_type=jax.ShapeDtypeStruct((n_points, D), jnp.float32),
        scratch_types=[
            pltpu.VMEM((ppw * 3,), jnp.int32),
            pltpu.VMEM((_CH * 3, 16), jnp.float32),
            pltpu.VMEM((_CH * 3, D), jnp.float32),
            pltpu.VMEM((_CH, D), jnp.float32),
            pltpu.SemaphoreType.DMA,
        ],
    )
    def k(table_hbm, idx_hbm, w_hbm, out_hbm, idx_v, w_v, rows_v, acc_v, sem):
        wid = lax.axis_index("s") * NC + lax.axis_index("c")
        pbase = wid * ppw
        pltpu.sync_copy(idx_hbm.at[pl.ds(pbase * 3, ppw * 3)], idx_v)

        def chunk(g, _):
            pltpu.sync_copy(
                w_hbm.at[pl.ds((pbase + g * _CH) * 3, _CH * 3)], w_v)
            pltpu.async_copy(
                table_hbm.at[idx_v.at[pl.ds(g * (_CH * 3), _CH * 3)]],
                rows_v, sem).wait()
            for p in range(_CH):
                ws = [w_v[3 * p + kk, :] for kk in range(3)]
                for sl in range(D // 16):
                    s = pl.ds(sl * 16, 16)
                    acc_v[p, s] = (ws[0] * rows_v[3 * p, s]
                                   + ws[1] * rows_v[3 * p + 1, s]
                                   + ws[2] * rows_v[3 * p + 2, s])
            pltpu.sync_copy(acc_v, out_hbm.at[pl.ds(pbase + g * _CH, _CH)])
            return 0

        lax.fori_loop(0, nch, chunk, 0)

    return k


def _mlp_body(interp_ref, p2_ref, xyz2_ref, w1a_ref, w1b_ref, w2_ref,
              b1_ref, b2_ref, out_ref):
    interp = interp_ref[0].astype(jnp.bfloat16)
    h1 = jnp.dot(interp, w1a_ref[...], preferred_element_type=jnp.float32)
    h1 += jnp.dot(p2_ref[0], w1b_ref[...], preferred_element_type=jnp.float32)
    h1 = jnp.maximum(h1 + b1_ref[...], 0.0).astype(jnp.bfloat16)
    h2 = jnp.dot(h1, w2_ref[...], preferred_element_type=jnp.float32)
    h2 = jnp.maximum(h2 + b2_ref[...], 0.0)
    out_ref[0] = jnp.concatenate([xyz2_ref[0], h2], axis=1)


@jax.jit
def _mlp_tc(interp, points2, xyz2, W1a, W1b, W2, b1, b2, blk=1024):
    B, N2, _ = points2.shape
    return pl.pallas_call(
        _mlp_body,
        grid=(B, N2 // blk),
        in_specs=[
            pl.BlockSpec((1, blk, 512), lambda b, j: (b, j, 0)),
            pl.BlockSpec((1, blk, 256), lambda b, j: (b, j, 0)),
            pl.BlockSpec((1, blk, 3), lambda b, j: (b, j, 0)),
            pl.BlockSpec((512, 512), lambda b, j: (0, 0)),
            pl.BlockSpec((256, 512), lambda b, j: (0, 0)),
            pl.BlockSpec((512, 512), lambda b, j: (0, 0)),
            pl.BlockSpec((1, 512), lambda b, j: (0, 0)),
            pl.BlockSpec((1, 512), lambda b, j: (0, 0)),
        ],
        out_specs=pl.BlockSpec((1, blk, 515), lambda b, j: (b, j, 0)),
        out_shape=jax.ShapeDtypeStruct((B, N2, 515), jnp.float32),
        compiler_params=pltpu.CompilerParams(
            dimension_semantics=("arbitrary", "arbitrary"),
        ),
    )(interp, points2, xyz2, W1a, W1b, W2, b1, b2)


def kernel(inputs_0, inputs_1, W1, b1, W2, b2):
    xyz1 = inputs_0[:, :, 0:3]
    points1 = inputs_0[:, :, 3:]
    xyz2 = inputs_1[:, :, 0:3]
    points2 = inputs_1[:, :, 3:]
    B, N2 = points2.shape[0], points2.shape[1]
    xyz1t = jnp.transpose(xyz1, (0, 2, 1))

    iw = _three_nn_tc(xyz1t, xyz2)
    idx_flat = iw[:, :, 0:3].reshape(-1).astype(jnp.int32)
    w_flat = iw[:, :, 3:6].reshape(-1)
    w_rep = jnp.broadcast_to(w_flat[:, None], (w_flat.shape[0], 16))
    w_rep = jnp.asarray(w_rep)
    table = points1.reshape(B * points1.shape[1], 512)

    sc_interp = _make_sc_interp(B * N2, 512)
    interp = sc_interp(table, idx_flat, w_rep).reshape(B, N2, 512)

    W1a = W1[:512, :].astype(jnp.bfloat16)
    W1b = W1[512:, :].astype(jnp.bfloat16)
    new_points = _mlp_tc(interp, points2.astype(jnp.bfloat16), xyz2,
                         W1a, W1b, W2.astype(jnp.bfloat16),
                         b1.reshape(1, -1), b2.reshape(1, -1))
    return (new_points, xyz2)


# speedup vs baseline: 4.2721x; 4.2721x over previous
import functools

import jax
import jax.numpy as jnp
from jax.experimental import pallas as pl
from jax.experimental.pallas import tpu as pltpu

EPS = 1e-7
BIG = 3e38


def _fused_body(xyz1t_ref, p1_ref, w1a_ref, w1b_ref, w2_ref, b1_ref, b2_ref,
                xyz2_ref, p2_ref, out_ref, pw_scratch):
    j = pl.program_id(1)

    @pl.when(j == 0)
    def _():
        pw_scratch[...] = jnp.dot(p1_ref[0], w1a_ref[...],
                                  preferred_element_type=jnp.float32
                                  ).astype(jnp.bfloat16)

    x1t = xyz1t_ref[0]
    sq1 = jnp.sum(x1t * x1t, axis=0, keepdims=True)
    x2 = xyz2_ref[0]
    sq2 = jnp.sum(x2 * x2, axis=1, keepdims=True)
    inner = jnp.dot(x2, x1t, preferred_element_type=jnp.float32)
    t = sq1 - 2.0 * inner

    m1 = jnp.min(t, axis=1, keepdims=True)
    t2 = jnp.where(t == m1, BIG, t)
    m2 = jnp.min(t2, axis=1, keepdims=True)
    m3 = jnp.min(jnp.where(t2 == m2, BIG, t2), axis=1, keepdims=True)

    d = [jnp.maximum(v + sq2, EPS) for v in (m1, m2, m3)]
    w = [1.0 / dk for dk in d]
    norm = w[0] + w[1] + w[2]
    w = [wk / norm for wk in w]

    zero = jnp.zeros_like(t)
    A = jnp.where(t == m1, w[0],
                  jnp.where(t == m2, w[1],
                            jnp.where(t == m3, w[2], zero))
                  ).astype(jnp.bfloat16)

    h1 = jnp.dot(A, pw_scratch[...], preferred_element_type=jnp.float32)
    h1 += jnp.dot(p2_ref[0], w1b_ref[...], preferred_element_type=jnp.float32)
    h1 = jnp.maximum(h1 + b1_ref[...], 0.0).astype(jnp.bfloat16)
    h2 = jnp.dot(h1, w2_ref[...], preferred_element_type=jnp.float32)
    h2 = jnp.maximum(h2 + b2_ref[...], 0.0)
    out_ref[0] = jnp.concatenate([x2, h2], axis=1)


@functools.partial(jax.jit, static_argnames=("blk",))
def _fused(xyz1t, points1, xyz2, points2, W1a, W1b, W2, b1, b2, blk=1024):
    B, N2, _ = points2.shape
    N1 = points1.shape[1]
    C1 = points1.shape[2]
    grid = (B, N2 // blk)
    return pl.pallas_call(
        _fused_body,
        grid=grid,
        in_specs=[
            pl.BlockSpec((1, 3, N1), lambda b, j: (b, 0, 0)),
            pl.BlockSpec((1, N1, C1), lambda b, j: (b, 0, 0)),
            pl.BlockSpec((C1, 512), lambda b, j: (0, 0)),
            pl.BlockSpec((256, 512), lambda b, j: (0, 0)),
            pl.BlockSpec((512, 512), lambda b, j: (0, 0)),
            pl.BlockSpec((1, 512), lambda b, j: (0, 0)),
            pl.BlockSpec((1, 512), lambda b, j: (0, 0)),
            pl.BlockSpec((1, blk, 3), lambda b, j: (b, j, 0)),
            pl.BlockSpec((1, blk, 256), lambda b, j: (b, j, 0)),
        ],
        out_specs=pl.BlockSpec((1, blk, 515), lambda b, j: (b, j, 0)),
        out_shape=jax.ShapeDtypeStruct((B, N2, 515), jnp.float32),
        scratch_shapes=[pltpu.VMEM((N1, 512), jnp.bfloat16)],
        compiler_params=pltpu.CompilerParams(
            dimension_semantics=("arbitrary", "arbitrary"),
        ),
    )(xyz1t, points1, W1a, W1b, W2, b1, b2, xyz2, points2)


def kernel(inputs_0, inputs_1, W1, b1, W2, b2):
    xyz1 = inputs_0[:, :, 0:3]
    points1 = inputs_0[:, :, 3:]
    xyz2 = inputs_1[:, :, 0:3]
    points2 = inputs_1[:, :, 3:]
    xyz1t = jnp.transpose(xyz1, (0, 2, 1))
    W1a = W1[:512, :].astype(jnp.bfloat16)
    W1b = W1[512:, :].astype(jnp.bfloat16)
    new_points = _fused(xyz1t, points1.astype(jnp.bfloat16),
                        xyz2, points2.astype(jnp.bfloat16),
                        W1a, W1b, W2.astype(jnp.bfloat16),
                        b1.reshape(1, -1), b2.reshape(1, -1))
    return (new_points, xyz2)
